# Initial kernel scaffold; baseline (speedup 1.0000x reference)
import jax
import jax.numpy as jnp
from jax.experimental import pallas as pl


def kernel(xyz, params):
    raise NotImplementedError("write your pallas kernel here")


# speedup vs baseline: 2.7098x; 2.7098x over previous
import functools

import jax
import jax.numpy as jnp
import numpy as np
from jax.experimental import pallas as pl

_NEG = -1e30


def _bf16rn(x):
    u = jax.lax.bitcast_convert_type(x, jnp.uint32)
    lsb = jax.lax.shift_right_logical(u, jnp.uint32(16)) & jnp.uint32(1)
    r = (u + jnp.uint32(0x7FFF) + lsb) & jnp.uint32(0xFFFF0000)
    return jax.lax.bitcast_convert_type(r, jnp.float32)


def _fps_body(xs_ref, ys_ref, zs_ref, cx_ref, cy_ref, cz_ref, *, npoint):
    B, N = xs_ref.shape
    xs = xs_ref[:, :]
    ys = ys_ref[:, :]
    zs = zs_ref[:, :]
    iota_n = jax.lax.broadcasted_iota(jnp.int32, (B, N), 1)
    iota_s = jax.lax.broadcasted_iota(jnp.int32, (B, npoint), 1)

    def body(i, carry):
        dist, far, cxa, cya, cza = carry
        sel = iota_n == far
        cxv = jnp.sum(jnp.where(sel, xs, 0.0), axis=1, keepdims=True)
        cyv = jnp.sum(jnp.where(sel, ys, 0.0), axis=1, keepdims=True)
        czv = jnp.sum(jnp.where(sel, zs, 0.0), axis=1, keepdims=True)
        cxa = jnp.where(iota_s == i, cxv, cxa)
        cya = jnp.where(iota_s == i, cyv, cya)
        cza = jnp.where(iota_s == i, czv, cza)
        d = (xs - cxv) ** 2 + (ys - cyv) ** 2 + (zs - czv) ** 2
        dist = jnp.minimum(dist, d)
        rm = jnp.max(dist, axis=1, keepdims=True)
        far = jnp.min(jnp.where(dist == rm, iota_n, N), axis=1, keepdims=True)
        return dist, far, cxa, cya, cza

    dist0 = jnp.full((B, N), 1e10, jnp.float32)
    far0 = jnp.zeros((B, 1), jnp.int32)
    acc0 = jnp.zeros((B, npoint), jnp.float32)
    _, _, cxa, cya, cza = jax.lax.fori_loop(
        0, npoint, body, (dist0, far0, acc0, acc0, acc0))
    cx_ref[:, :] = cxa
    cy_ref[:, :] = cya
    cz_ref[:, :] = cza


def _fps(xs, ys, zs, npoint):
    B, N = xs.shape
    out = jax.ShapeDtypeStruct((B, npoint), jnp.float32)
    return pl.pallas_call(
        functools.partial(_fps_body, npoint=npoint),
        out_shape=(out, out, out),
    )(xs, ys, zs)


def _feat_proj_body(f_ref, w0_ref, w1_ref, w2_ref, o0_ref, o1_ref, o2_ref):
    f = f_ref[0]
    o0_ref[0] = jnp.dot(f, w0_ref[:, :], preferred_element_type=jnp.float32, precision=jax.lax.Precision.HIGHEST)
    o1_ref[0] = jnp.dot(f, w1_ref[:, :], preferred_element_type=jnp.float32, precision=jax.lax.Precision.HIGHEST)
    o2_ref[0] = jnp.dot(f, w2_ref[:, :], preferred_element_type=jnp.float32, precision=jax.lax.Precision.HIGHEST)


def _feat_proj(feats, w0, w1, w2):
    B, N, _ = feats.shape
    outs = tuple(jax.ShapeDtypeStruct((B, N, w.shape[1]), jnp.float32)
                 for w in (w0, w1, w2))
    full = lambda s: pl.BlockSpec(s, lambda b: (0,) * len(s))
    return pl.pallas_call(
        _feat_proj_body,
        grid=(B,),
        in_specs=[
            pl.BlockSpec((1, N, feats.shape[2]), lambda b: (b, 0, 0)),
            full(w0.shape), full(w1.shape), full(w2.shape),
        ],
        out_specs=tuple(
            pl.BlockSpec((1, N, w.shape[1]), lambda b: (b, 0, 0))
            for w in (w0, w1, w2)),
        out_shape=outs,
    )(feats, w0, w1, w2)


def _branch_body(xs_ref, ys_ref, zs_ref, xyzc_ref, cx_ref, cy_ref, cz_ref,
                 u_ref, f1_ref, w1x_ref, b1_ref, w2_ref, b2_ref, w3_ref,
                 b3_ref, out_ref, *, r2, K, s_blk):
    N = xs_ref.shape[2]
    nc = N // 128
    xs = xs_ref[0]
    ys = ys_ref[0]
    zs = zs_ref[0]
    cx = cx_ref[0, 0]
    cy = cy_ref[0, 0]
    cz = cz_ref[0, 0]
    pn2 = xs * xs + ys * ys + zs * zs
    cn2 = cx * cx + cy * cy + cz * cz
    xb, yb, zb = _bf16rn(xs), _bf16rn(ys), _bf16rn(zs)
    cxb, cyb, czb = _bf16rn(cx), _bf16rn(cy), _bf16rn(cz)
    dot = cxb * xb + cyb * yb + czb * zb
    sq = cn2 + pn2 - 2.0 * dot
    valid = sq <= r2
    vf = jnp.where(valid, 1.0, 0.0)
    u = u_ref[:, :]
    carry = jnp.zeros((s_blk, 1), jnp.float32)
    pieces = []
    for j in range(nc):
        mj = vf[:, j * 128:(j + 1) * 128]
        inc = jnp.dot(mj, u, preferred_element_type=jnp.float32, precision=jax.lax.Precision.HIGHEST)
        pieces.append(inc - mj + carry)
        carry = carry + inc[:, 127:128]
    rank = jnp.concatenate(pieces, axis=1)
    cnt = carry
    rankm = jnp.where(valid, rank, -1.0).astype(jnp.int32)
    nio = jax.lax.broadcasted_iota(jnp.int32, (s_blk, N), 1)
    rankm = jnp.where((cnt == 0.0) & (nio == N - 1), 0, rankm)
    kio = jax.lax.broadcasted_iota(jnp.int32, (s_blk, K, N), 1)
    p = jnp.where(rankm[:, None, :] == kio, 1.0, 0.0).reshape(s_blk * K, N)
    g3 = jnp.dot(p, xyzc_ref[0], preferred_element_type=jnp.float32, precision=jax.lax.Precision.HIGHEST)
    c3 = jnp.concatenate([cx, cy, cz], axis=1)
    crep = jnp.broadcast_to(c3[:, None, :], (s_blk, K, 3)).reshape(s_blk * K, 3)
    dx = g3 - crep
    h = jnp.dot(dx, w1x_ref[:, :], preferred_element_type=jnp.float32, precision=jax.lax.Precision.HIGHEST)
    if f1_ref is not None:
        h = h + jnp.dot(p, f1_ref[0], preferred_element_type=jnp.float32, precision=jax.lax.Precision.HIGHEST)
    h = jnp.maximum(h + b1_ref[:, :], 0.0)
    h = jnp.maximum(jnp.dot(h, w2_ref[:, :], preferred_element_type=jnp.float32, precision=jax.lax.Precision.HIGHEST)
                    + b2_ref[:, :], 0.0)
    h = jnp.maximum(jnp.dot(h, w3_ref[:, :], preferred_element_type=jnp.float32, precision=jax.lax.Precision.HIGHEST)
                    + b3_ref[:, :], 0.0)
    c_out = h.shape[1]
    h3 = h.reshape(s_blk, K, c_out)
    slot = jax.lax.broadcasted_iota(jnp.int32, (s_blk, K, 1), 1)
    cnt_eff = jnp.maximum(cnt[:, :, None].astype(jnp.int32), 1)
    hm = jnp.where(slot < cnt_eff, h3, _NEG)
    out_ref[0] = jnp.max(hm, axis=1)


def _branch(xs, ys, zs, xyzc, cx, cy, cz, u128, f1, w1x, b1, w2, b2,
            w3, b3, radius, K, s_blk):
    B, N = xs.shape
    S = cx.shape[1]
    nt = S // s_blk
    c3 = w3.shape[1]
    xs3 = xs.reshape(B, 1, N)
    ys3 = ys.reshape(B, 1, N)
    zs3 = zs.reshape(B, 1, N)
    cx4 = cx.reshape(B, nt, s_blk, 1)
    cy4 = cy.reshape(B, nt, s_blk, 1)
    cz4 = cz.reshape(B, nt, s_blk, 1)
    kw = dict(r2=radius * radius, K=K, s_blk=s_blk)
    if f1 is not None:
        body = functools.partial(_branch_body, **kw)
    else:
        def body(*refs):
            _branch_body(*refs[:8], None, *refs[8:], **kw)
    bn = lambda s: pl.BlockSpec(s, lambda b, j: (0,) * len(s))
    in_specs = [
        pl.BlockSpec((1, 1, N), lambda b, j: (b, 0, 0)),
        pl.BlockSpec((1, 1, N), lambda b, j: (b, 0, 0)),
        pl.BlockSpec((1, 1, N), lambda b, j: (b, 0, 0)),
        pl.BlockSpec((1, N, 3), lambda b, j: (b, 0, 0)),
        pl.BlockSpec((1, 1, s_blk, 1), lambda b, j: (b, j, 0, 0)),
        pl.BlockSpec((1, 1, s_blk, 1), lambda b, j: (b, j, 0, 0)),
        pl.BlockSpec((1, 1, s_blk, 1), lambda b, j: (b, j, 0, 0)),
        bn(u128.shape),
    ]
    args = [xs3, ys3, zs3, xyzc, cx4, cy4, cz4, u128]
    if f1 is not None:
        in_specs.append(pl.BlockSpec((1, N, f1.shape[2]),
                                     lambda b, j: (b, 0, 0)))
        args.append(f1)
    in_specs += [bn(w1x.shape), bn(b1.shape), bn(w2.shape), bn(b2.shape),
                 bn(w3.shape), bn(b3.shape)]
    args += [w1x, b1, w2, b2, w3, b3]
    return pl.pallas_call(
        body,
        grid=(B, S // s_blk),
        in_specs=in_specs,
        out_specs=pl.BlockSpec((1, s_blk, c3), lambda b, j: (b, j, 0)),
        out_shape=jax.ShapeDtypeStruct((B, S, c3), jnp.float32),
    )(*args)


_RADII = [[0.1, 0.2, 0.4], [0.2, 0.4, 0.8]]
_KS = [[8, 16, 32], [16, 32, 64]]
_RATIO = 0.25
_SBLK = [8, 16]


def kernel(xyz, params):
    u128 = jnp.asarray(np.triu(np.ones((128, 128), np.float32)))
    feats = None
    f1s = [None, None, None]
    for mod in range(2):
        B, N, _ = xyz.shape
        S = int(N * _RATIO)
        xs, ys, zs = (xyz[:, :, i] for i in range(3))
        cx, cy, cz = _fps(xs, ys, zs, S)
        if feats is not None:
            ws = [p[0][0] for p in params[mod]]
            f1s = list(_feat_proj(feats, *[w[:-3] for w in ws]))
        outs = []
        for br in range(3):
            layers = params[mod][br]
            (w1, b1), (w2, b2), (w3, b3) = layers
            outs.append(_branch(
                xs, ys, zs, xyz, cx, cy, cz, u128, f1s[br],
                w1[-3:], b1[None, :], w2, b2[None, :], w3, b3[None, :],
                _RADII[mod][br], _KS[mod][br], _SBLK[mod]))
        feats = jnp.concatenate(outs, axis=-1)
        xyz = jnp.stack([cx, cy, cz], axis=-1)
    return feats


# speedup vs baseline: 13.3112x; 4.9122x over previous
import functools

import jax
import jax.numpy as jnp
import numpy as np
from jax.experimental import pallas as pl

_NEG = -1e30


def _bf16rn(x):
    u = jax.lax.bitcast_convert_type(x, jnp.uint32)
    lsb = jax.lax.shift_right_logical(u, jnp.uint32(16)) & jnp.uint32(1)
    r = (u + jnp.uint32(0x7FFF) + lsb) & jnp.uint32(0xFFFF0000)
    return jax.lax.bitcast_convert_type(r, jnp.float32)


def _fps_body(xs_ref, ys_ref, zs_ref, cx_ref, cy_ref, cz_ref, *, npoint):
    B, N = xs_ref.shape
    xs = xs_ref[:, :]
    ys = ys_ref[:, :]
    zs = zs_ref[:, :]
    iota_n = jax.lax.broadcasted_iota(jnp.int32, (B, N), 1)
    iota_s = jax.lax.broadcasted_iota(jnp.int32, (B, npoint), 1)

    def body(i, carry):
        dist, far, cxa, cya, cza = carry
        sel = iota_n == far
        cxv = jnp.sum(jnp.where(sel, xs, 0.0), axis=1, keepdims=True)
        cyv = jnp.sum(jnp.where(sel, ys, 0.0), axis=1, keepdims=True)
        czv = jnp.sum(jnp.where(sel, zs, 0.0), axis=1, keepdims=True)
        cxa = jnp.where(iota_s == i, cxv, cxa)
        cya = jnp.where(iota_s == i, cyv, cya)
        cza = jnp.where(iota_s == i, czv, cza)
        d = (xs - cxv) ** 2 + (ys - cyv) ** 2 + (zs - czv) ** 2
        dist = jnp.minimum(dist, d)
        rm = jnp.max(dist, axis=1, keepdims=True)
        far = jnp.min(jnp.where(dist == rm, iota_n, N), axis=1, keepdims=True)
        return dist, far, cxa, cya, cza

    dist0 = jnp.full((B, N), 1e10, jnp.float32)
    far0 = jnp.zeros((B, 1), jnp.int32)
    acc0 = jnp.zeros((B, npoint), jnp.float32)
    _, _, cxa, cya, cza = jax.lax.fori_loop(
        0, npoint, body, (dist0, far0, acc0, acc0, acc0))
    cx_ref[:, :] = cxa
    cy_ref[:, :] = cya
    cz_ref[:, :] = cza


def _fps(xs, ys, zs, npoint):
    B, N = xs.shape
    out = jax.ShapeDtypeStruct((B, npoint), jnp.float32)
    return pl.pallas_call(
        functools.partial(_fps_body, npoint=npoint),
        out_shape=(out, out, out),
    )(xs, ys, zs)


def _feat_proj_body(f_ref, w0_ref, w1_ref, w2_ref, o0_ref, o1_ref, o2_ref):
    f = f_ref[0]
    o0_ref[0] = jnp.dot(f, w0_ref[:, :], preferred_element_type=jnp.float32)
    o1_ref[0] = jnp.dot(f, w1_ref[:, :], preferred_element_type=jnp.float32)
    o2_ref[0] = jnp.dot(f, w2_ref[:, :], preferred_element_type=jnp.float32)


def _feat_proj(feats, w0, w1, w2):
    B, N, _ = feats.shape
    outs = tuple(jax.ShapeDtypeStruct((B, N, w.shape[1]), jnp.float32)
                 for w in (w0, w1, w2))
    full = lambda s: pl.BlockSpec(s, lambda b: (0,) * len(s))
    return pl.pallas_call(
        _feat_proj_body,
        grid=(B,),
        in_specs=[
            pl.BlockSpec((1, N, feats.shape[2]), lambda b: (b, 0, 0)),
            full(w0.shape), full(w1.shape), full(w2.shape),
        ],
        out_specs=tuple(
            pl.BlockSpec((1, N, w.shape[1]), lambda b: (b, 0, 0))
            for w in (w0, w1, w2)),
        out_shape=outs,
    )(feats, w0, w1, w2)


def _branch_body(xs_ref, ys_ref, zs_ref, xyzc_ref, cx_ref, cy_ref, cz_ref,
                 u_ref, f1_ref, w1x_ref, b1_ref, w2_ref, b2_ref, w3_ref,
                 b3_ref, out_ref, *, r2, K, s_blk):
    N = xs_ref.shape[2]
    nc = N // 128
    xs = xs_ref[0]
    ys = ys_ref[0]
    zs = zs_ref[0]
    cx = cx_ref[0, 0]
    cy = cy_ref[0, 0]
    cz = cz_ref[0, 0]
    pn2 = xs * xs + ys * ys + zs * zs
    cn2 = cx * cx + cy * cy + cz * cz
    xb, yb, zb = _bf16rn(xs), _bf16rn(ys), _bf16rn(zs)
    cxb, cyb, czb = _bf16rn(cx), _bf16rn(cy), _bf16rn(cz)
    dot = cxb * xb + cyb * yb + czb * zb
    sq = cn2 + pn2 - 2.0 * dot
    valid = sq <= r2
    vf = jnp.where(valid, 1.0, 0.0)
    u = u_ref[:, :]
    carry = jnp.zeros((s_blk, 1), jnp.float32)
    pieces = []
    for j in range(nc):
        mj = vf[:, j * 128:(j + 1) * 128]
        inc = jnp.dot(mj, u, preferred_element_type=jnp.float32)
        pieces.append(inc - mj + carry)
        carry = carry + inc[:, 127:128]
    rank = jnp.concatenate(pieces, axis=1)
    cnt = carry
    rankm = jnp.where(valid, rank, -1.0).astype(jnp.int32)
    nio = jax.lax.broadcasted_iota(jnp.int32, (s_blk, N), 1)
    rankm = jnp.where((cnt == 0.0) & (nio == N - 1), 0, rankm)
    kio = jax.lax.broadcasted_iota(jnp.int32, (s_blk, K, N), 1)
    p = jnp.where(rankm[:, None, :] == kio, 1.0, 0.0).reshape(s_blk * K, N)
    g3 = jnp.dot(p, xyzc_ref[0], preferred_element_type=jnp.float32)
    c3 = jnp.concatenate([cx, cy, cz], axis=1)
    crep = jnp.broadcast_to(c3[:, None, :], (s_blk, K, 3)).reshape(s_blk * K, 3)
    dx = g3 - crep
    h = jnp.dot(dx, w1x_ref[:, :], preferred_element_type=jnp.float32)
    if f1_ref is not None:
        h = h + jnp.dot(p, f1_ref[0], preferred_element_type=jnp.float32)
    h = jnp.maximum(h + b1_ref[:, :], 0.0)
    h = jnp.maximum(jnp.dot(h, w2_ref[:, :], preferred_element_type=jnp.float32)
                    + b2_ref[:, :], 0.0)
    h = jnp.maximum(jnp.dot(h, w3_ref[:, :], preferred_element_type=jnp.float32)
                    + b3_ref[:, :], 0.0)
    c_out = h.shape[1]
    h3 = h.reshape(s_blk, K, c_out)
    slot = jax.lax.broadcasted_iota(jnp.int32, (s_blk, K, 1), 1)
    cnt_eff = jnp.maximum(cnt[:, :, None].astype(jnp.int32), 1)
    hm = jnp.where(slot < cnt_eff, h3, _NEG)
    out_ref[0] = jnp.max(hm, axis=1)


def _branch(xs, ys, zs, xyzc, cx, cy, cz, u128, f1, w1x, b1, w2, b2,
            w3, b3, radius, K, s_blk):
    B, N = xs.shape
    S = cx.shape[1]
    nt = S // s_blk
    c3 = w3.shape[1]
    xs3 = xs.reshape(B, 1, N)
    ys3 = ys.reshape(B, 1, N)
    zs3 = zs.reshape(B, 1, N)
    cx4 = cx.reshape(B, nt, s_blk, 1)
    cy4 = cy.reshape(B, nt, s_blk, 1)
    cz4 = cz.reshape(B, nt, s_blk, 1)
    kw = dict(r2=radius * radius, K=K, s_blk=s_blk)
    if f1 is not None:
        body = functools.partial(_branch_body, **kw)
    else:
        def body(*refs):
            _branch_body(*refs[:8], None, *refs[8:], **kw)
    bn = lambda s: pl.BlockSpec(s, lambda b, j: (0,) * len(s))
    in_specs = [
        pl.BlockSpec((1, 1, N), lambda b, j: (b, 0, 0)),
        pl.BlockSpec((1, 1, N), lambda b, j: (b, 0, 0)),
        pl.BlockSpec((1, 1, N), lambda b, j: (b, 0, 0)),
        pl.BlockSpec((1, N, 3), lambda b, j: (b, 0, 0)),
        pl.BlockSpec((1, 1, s_blk, 1), lambda b, j: (b, j, 0, 0)),
        pl.BlockSpec((1, 1, s_blk, 1), lambda b, j: (b, j, 0, 0)),
        pl.BlockSpec((1, 1, s_blk, 1), lambda b, j: (b, j, 0, 0)),
        bn(u128.shape),
    ]
    args = [xs3, ys3, zs3, xyzc, cx4, cy4, cz4, u128]
    if f1 is not None:
        in_specs.append(pl.BlockSpec((1, N, f1.shape[2]),
                                     lambda b, j: (b, 0, 0)))
        args.append(f1)
    in_specs += [bn(w1x.shape), bn(b1.shape), bn(w2.shape), bn(b2.shape),
                 bn(w3.shape), bn(b3.shape)]
    args += [w1x, b1, w2, b2, w3, b3]
    return pl.pallas_call(
        body,
        grid=(B, S // s_blk),
        in_specs=in_specs,
        out_specs=pl.BlockSpec((1, s_blk, c3), lambda b, j: (b, j, 0)),
        out_shape=jax.ShapeDtypeStruct((B, S, c3), jnp.float32),
    )(*args)


_RADII = [[0.1, 0.2, 0.4], [0.2, 0.4, 0.8]]
_KS = [[8, 16, 32], [16, 32, 64]]
_RATIO = 0.25
_SBLK = [16, 32]


def kernel(xyz, params):
    u128 = jnp.asarray(np.triu(np.ones((128, 128), np.float32)))
    feats = None
    f1s = [None, None, None]
    for mod in range(2):
        B, N, _ = xyz.shape
        S = int(N * _RATIO)
        xs, ys, zs = (xyz[:, :, i] for i in range(3))
        cx, cy, cz = _fps(xs, ys, zs, S)
        if feats is not None:
            ws = [p[0][0] for p in params[mod]]
            f1s = list(_feat_proj(feats, *[w[:-3] for w in ws]))
        outs = []
        for br in range(3):
            layers = params[mod][br]
            (w1, b1), (w2, b2), (w3, b3) = layers
            outs.append(_branch(
                xs, ys, zs, xyz, cx, cy, cz, u128, f1s[br],
                w1[-3:], b1[None, :], w2, b2[None, :], w3, b3[None, :],
                _RADII[mod][br], _KS[mod][br], _SBLK[mod]))
        feats = jnp.concatenate(outs, axis=-1)
        xyz = jnp.stack([cx, cy, cz], axis=-1)
    return feats


# speedup vs baseline: 16.7031x; 1.2548x over previous
import functools

import jax
import jax.numpy as jnp
import numpy as np
from jax.experimental import pallas as pl

_NEG = -1e30


def _bf16rn(x):
    u = jax.lax.bitcast_convert_type(x, jnp.uint32)
    lsb = jax.lax.shift_right_logical(u, jnp.uint32(16)) & jnp.uint32(1)
    r = (u + jnp.uint32(0x7FFF) + lsb) & jnp.uint32(0xFFFF0000)
    return jax.lax.bitcast_convert_type(r, jnp.float32)


def _fps_body(xs_ref, ys_ref, zs_ref, cx_ref, cy_ref, cz_ref, *, npoint):
    B, N = xs_ref.shape
    xs = xs_ref[:, :]
    ys = ys_ref[:, :]
    zs = zs_ref[:, :]
    iota_n = jax.lax.broadcasted_iota(jnp.int32, (B, N), 1)
    iota_s = jax.lax.broadcasted_iota(jnp.int32, (B, npoint), 1)

    def body(i, carry):
        dist, far, cxa, cya, cza = carry
        sel = iota_n == far
        cxv = jnp.sum(jnp.where(sel, xs, 0.0), axis=1, keepdims=True)
        cyv = jnp.sum(jnp.where(sel, ys, 0.0), axis=1, keepdims=True)
        czv = jnp.sum(jnp.where(sel, zs, 0.0), axis=1, keepdims=True)
        cxa = jnp.where(iota_s == i, cxv, cxa)
        cya = jnp.where(iota_s == i, cyv, cya)
        cza = jnp.where(iota_s == i, czv, cza)
        d = (xs - cxv) ** 2 + (ys - cyv) ** 2 + (zs - czv) ** 2
        dist = jnp.minimum(dist, d)
        rm = jnp.max(dist, axis=1, keepdims=True)
        far = jnp.min(jnp.where(dist == rm, iota_n, N), axis=1, keepdims=True)
        return dist, far, cxa, cya, cza

    dist0 = jnp.full((B, N), 1e10, jnp.float32)
    far0 = jnp.zeros((B, 1), jnp.int32)
    acc0 = jnp.zeros((B, npoint), jnp.float32)
    _, _, cxa, cya, cza = jax.lax.fori_loop(
        0, npoint, body, (dist0, far0, acc0, acc0, acc0))
    cx_ref[:, :] = cxa
    cy_ref[:, :] = cya
    cz_ref[:, :] = cza


def _fps(xs, ys, zs, npoint):
    B, N = xs.shape
    out = jax.ShapeDtypeStruct((B, npoint), jnp.float32)
    return pl.pallas_call(
        functools.partial(_fps_body, npoint=npoint),
        out_shape=(out, out, out),
    )(xs, ys, zs)


def _feat_proj_body(f_ref, w0_ref, w1_ref, w2_ref, o0_ref, o1_ref, o2_ref):
    f = f_ref[0]
    o0_ref[0] = jnp.dot(f, w0_ref[:, :], preferred_element_type=jnp.float32)
    o1_ref[0] = jnp.dot(f, w1_ref[:, :], preferred_element_type=jnp.float32)
    o2_ref[0] = jnp.dot(f, w2_ref[:, :], preferred_element_type=jnp.float32)


def _feat_proj(feats, w0, w1, w2):
    B, N, _ = feats.shape
    outs = tuple(jax.ShapeDtypeStruct((B, N, w.shape[1]), jnp.float32)
                 for w in (w0, w1, w2))
    full = lambda s: pl.BlockSpec(s, lambda b: (0,) * len(s))
    return pl.pallas_call(
        _feat_proj_body,
        grid=(B,),
        in_specs=[
            pl.BlockSpec((1, N, feats.shape[2]), lambda b: (b, 0, 0)),
            full(w0.shape), full(w1.shape), full(w2.shape),
        ],
        out_specs=tuple(
            pl.BlockSpec((1, N, w.shape[1]), lambda b: (b, 0, 0))
            for w in (w0, w1, w2)),
        out_shape=outs,
    )(feats, w0, w1, w2)


def _branch_body(xs_ref, ys_ref, zs_ref, xyzc_ref, cx_ref, cy_ref, cz_ref,
                 u_ref, f1_ref, w1x_ref, b1_ref, w2_ref, b2_ref, w3_ref,
                 b3_ref, out_ref, *, r2, K, s_blk):
    N = xs_ref.shape[2]
    nc = N // 128
    xs = xs_ref[0]
    ys = ys_ref[0]
    zs = zs_ref[0]
    cx = cx_ref[0, 0]
    cy = cy_ref[0, 0]
    cz = cz_ref[0, 0]
    pn2 = xs * xs + ys * ys + zs * zs
    cn2 = cx * cx + cy * cy + cz * cz
    xb, yb, zb = _bf16rn(xs), _bf16rn(ys), _bf16rn(zs)
    cxb, cyb, czb = _bf16rn(cx), _bf16rn(cy), _bf16rn(cz)
    dot = cxb * xb + cyb * yb + czb * zb
    sq = cn2 + pn2 - 2.0 * dot
    valid = sq <= r2
    vf = jnp.where(valid, 1.0, 0.0)
    u = u_ref[:, :]
    carry = jnp.zeros((s_blk, 1), jnp.float32)
    pieces = []
    for j in range(nc):
        mj = vf[:, j * 128:(j + 1) * 128]
        inc = jnp.dot(mj, u, preferred_element_type=jnp.float32)
        pieces.append(inc - mj + carry)
        carry = carry + inc[:, 127:128]
    rank = jnp.concatenate(pieces, axis=1)
    cnt = carry
    rankm = jnp.where(valid, rank, -1.0).astype(jnp.int32)
    nio = jax.lax.broadcasted_iota(jnp.int32, (s_blk, N), 1)
    rankm = jnp.where((cnt == 0.0) & (nio == N - 1), 0, rankm)
    kio = jax.lax.broadcasted_iota(jnp.int32, (s_blk, K, N), 1)
    p = jnp.where(rankm[:, None, :] == kio, 1.0, 0.0).reshape(s_blk * K, N)
    g3 = jnp.dot(p, xyzc_ref[0], preferred_element_type=jnp.float32)
    c3 = jnp.concatenate([cx, cy, cz], axis=1)
    crep = jnp.broadcast_to(c3[:, None, :], (s_blk, K, 3)).reshape(s_blk * K, 3)
    dx = g3 - crep
    h = jnp.dot(dx, w1x_ref[:, :], preferred_element_type=jnp.float32)
    if f1_ref is not None:
        h = h + jnp.dot(p, f1_ref[0], preferred_element_type=jnp.float32)
    h = jnp.maximum(h + b1_ref[:, :], 0.0)
    h = jnp.maximum(jnp.dot(h, w2_ref[:, :], preferred_element_type=jnp.float32)
                    + b2_ref[:, :], 0.0)
    h = jnp.maximum(jnp.dot(h, w3_ref[:, :], preferred_element_type=jnp.float32)
                    + b3_ref[:, :], 0.0)
    c_out = h.shape[1]
    h3 = h.reshape(s_blk, K, c_out)
    slot = jax.lax.broadcasted_iota(jnp.int32, (s_blk, K, 1), 1)
    cnt_eff = jnp.maximum(cnt[:, :, None].astype(jnp.int32), 1)
    hm = jnp.where(slot < cnt_eff, h3, _NEG)
    out_ref[0] = jnp.max(hm, axis=1)


def _branch(xs, ys, zs, xyzc, cx, cy, cz, u128, f1, w1x, b1, w2, b2,
            w3, b3, radius, K, s_blk):
    B, N = xs.shape
    S = cx.shape[1]
    nt = S // s_blk
    c3 = w3.shape[1]
    xs3 = xs.reshape(B, 1, N)
    ys3 = ys.reshape(B, 1, N)
    zs3 = zs.reshape(B, 1, N)
    cx4 = cx.reshape(B, nt, s_blk, 1)
    cy4 = cy.reshape(B, nt, s_blk, 1)
    cz4 = cz.reshape(B, nt, s_blk, 1)
    kw = dict(r2=radius * radius, K=K, s_blk=s_blk)
    if f1 is not None:
        body = functools.partial(_branch_body, **kw)
    else:
        def body(*refs):
            _branch_body(*refs[:8], None, *refs[8:], **kw)
    bn = lambda s: pl.BlockSpec(s, lambda b, j: (0,) * len(s))
    in_specs = [
        pl.BlockSpec((1, 1, N), lambda b, j: (b, 0, 0)),
        pl.BlockSpec((1, 1, N), lambda b, j: (b, 0, 0)),
        pl.BlockSpec((1, 1, N), lambda b, j: (b, 0, 0)),
        pl.BlockSpec((1, N, 3), lambda b, j: (b, 0, 0)),
        pl.BlockSpec((1, 1, s_blk, 1), lambda b, j: (b, j, 0, 0)),
        pl.BlockSpec((1, 1, s_blk, 1), lambda b, j: (b, j, 0, 0)),
        pl.BlockSpec((1, 1, s_blk, 1), lambda b, j: (b, j, 0, 0)),
        bn(u128.shape),
    ]
    args = [xs3, ys3, zs3, xyzc, cx4, cy4, cz4, u128]
    if f1 is not None:
        in_specs.append(pl.BlockSpec((1, N, f1.shape[2]),
                                     lambda b, j: (b, 0, 0)))
        args.append(f1)
    in_specs += [bn(w1x.shape), bn(b1.shape), bn(w2.shape), bn(b2.shape),
                 bn(w3.shape), bn(b3.shape)]
    args += [w1x, b1, w2, b2, w3, b3]
    return pl.pallas_call(
        body,
        grid=(B, S // s_blk),
        in_specs=in_specs,
        out_specs=pl.BlockSpec((1, s_blk, c3), lambda b, j: (b, j, 0)),
        out_shape=jax.ShapeDtypeStruct((B, S, c3), jnp.float32),
    )(*args)


_RADII = [[0.1, 0.2, 0.4], [0.2, 0.4, 0.8]]
_KS = [[8, 16, 32], [16, 32, 64]]
_RATIO = 0.25
_SBLK = [32, 64]


def kernel(xyz, params):
    u128 = jnp.asarray(np.triu(np.ones((128, 128), np.float32)))
    feats = None
    f1s = [None, None, None]
    for mod in range(2):
        B, N, _ = xyz.shape
        S = int(N * _RATIO)
        xs, ys, zs = (xyz[:, :, i] for i in range(3))
        cx, cy, cz = _fps(xs, ys, zs, S)
        if feats is not None:
            ws = [p[0][0] for p in params[mod]]
            f1s = list(_feat_proj(feats, *[w[:-3] for w in ws]))
        outs = []
        for br in range(3):
            layers = params[mod][br]
            (w1, b1), (w2, b2), (w3, b3) = layers
            outs.append(_branch(
                xs, ys, zs, xyz, cx, cy, cz, u128, f1s[br],
                w1[-3:], b1[None, :], w2, b2[None, :], w3, b3[None, :],
                _RADII[mod][br], _KS[mod][br], _SBLK[mod]))
        feats = jnp.concatenate(outs, axis=-1)
        xyz = jnp.stack([cx, cy, cz], axis=-1)
    return feats
